# baseline (speedup 1.0000x reference)
import functools

import jax
import jax.numpy as jnp
from jax import lax
from jax.experimental import pallas as pl
from jax.experimental.pallas import tpu as pltpu
from jax.experimental.pallas import tpu_sc as plsc

VOCAB = 1000000
EMBED_DIM = 32
BATCH = 16384
HIST_LEN = 50

NC = 2
NS = 16
NW = NC * NS
NBUF = 4

NT_FULL = 7812
N_K1 = 244
NG1 = N_K1 // NBUF
K1_EXTRA = NT_FULL - NW * N_K1
TL_ROWS = 250000

NTB = BATCH // 128
TB_PER_W = NTB // NW
N_GROUPS = TB_PER_W * HIST_LEN
NG2 = N_GROUPS // NBUF

_MESH = plsc.VectorSubcoreMesh(core_axis_name="c", subcore_axis_name="s")


def _iota16():
    return lax.iota(jnp.int32, 16)


@functools.partial(
    pl.kernel,
    mesh=_MESH,
    out_type=jax.ShapeDtypeStruct((TL_ROWS * 128,), jnp.float32),
    scratch_types=(
        [pltpu.VMEM((32, 128), jnp.float32)] * NBUF
        + [pltpu.VMEM((4096,), jnp.float32)] * NBUF
        + [pltpu.SemaphoreType.DMA] * (2 * NBUF)
    ),
    compiler_params=pltpu.CompilerParams(needs_layout_passes=False),
)
def _transpose_kernel(tab_hbm, tail_hbm, tl_hbm, *bufs):
    tiles = bufs[:NBUF]
    stage = bufs[NBUF:2 * NBUF]
    rsem = bufs[2 * NBUF:3 * NBUF]
    wsem = bufs[3 * NBUF:]

    wid = lax.axis_index("s") * NC + lax.axis_index("c")

    def r_desc(t_v, b):
        return pltpu.make_async_copy(
            tab_hbm.at[:, pl.ds(t_v * 128, 128)], tiles[b], rsem[b])

    def w_desc(t_v, b):
        return pltpu.make_async_copy(
            stage[b], tl_hbm.at[pl.ds(t_v * 4096, 4096)], wsem[b])

    def extract(b):
        tl = tiles[b]
        st = stage[b]
        base = _iota16() * 32

        @plsc.parallel_loop(0, 8, unroll=2)
        def _(g):
            tgt = base + g * 512
            vals = [tl[d, pl.ds(g * 16, 16)] for d in range(EMBED_DIM)]
            for d in range(EMBED_DIM):
                plsc.store_scatter(st, [tgt + d], vals[d])

    def tv(i):
        return wid + NW * i

    for b in range(NBUF):
        r_desc(tv(b), b).start()

    def body(g, carry):
        for b in range(NBUF):
            i = g * NBUF + b
            r_desc(tv(i), b).wait()

            @pl.when(g >= 1)
            def _():
                w_desc(tv(i - NBUF), b).wait()

            extract(b)
            w_desc(tv(i), b).start()

            @pl.when(g < NG1 - 1)
            def _():
                r_desc(tv(i + NBUF), b).start()
        return carry

    lax.fori_loop(0, NG1, body, 0)

    t_extra = NW * N_K1 + wid
    w_desc(tv(N_K1 - NBUF), 0).wait()

    @pl.when(wid < K1_EXTRA)
    def _():
        r_desc(t_extra, 0).start()
        r_desc(t_extra, 0).wait()

    extract(0)

    @pl.when(wid < K1_EXTRA)
    def _():
        w_desc(t_extra, 0).start()
        w_desc(t_extra, 0).wait()

    @pl.when(wid == K1_EXTRA)
    def _():
        pltpu.sync_copy(tail_hbm, tl_hbm.at[pl.ds((TL_ROWS - 16) * 128, 2048)])

    for b in range(1, NBUF):
        w_desc(tv(N_K1 - NBUF + b), b).wait()


@functools.partial(
    pl.kernel,
    mesh=_MESH,
    out_type=jax.ShapeDtypeStruct((HIST_LEN, EMBED_DIM, BATCH), jnp.float32),
    scratch_types=(
        [pltpu.VMEM((128,), jnp.int32)] * NBUF
        + [pltpu.VMEM((128,), jnp.int32)] * NBUF
        + [pltpu.VMEM((128,), jnp.int32)] * NBUF
        + [pltpu.VMEM((128, 128), jnp.float32)] * NBUF
        + [pltpu.VMEM((32, 128), jnp.float32)] * NBUF
        + [pltpu.SemaphoreType.DMA] * (3 * NBUF)
    ),
    compiler_params=pltpu.CompilerParams(needs_layout_passes=False),
)
def _gather_kernel(bd_hbm, tl_hbm, out_hbm, *bufs):
    ibuf = bufs[:NBUF]
    gix = bufs[NBUF:2 * NBUF]
    cov = bufs[2 * NBUF:3 * NBUF]
    rows = bufs[3 * NBUF:4 * NBUF]
    stage = bufs[4 * NBUF:5 * NBUF]
    isem = bufs[5 * NBUF:6 * NBUF]
    gsem = bufs[6 * NBUF:7 * NBUF]
    osem = bufs[7 * NBUF:]

    wid = lax.axis_index("s") * NC + lax.axis_index("c")
    tb0 = wid * TB_PER_W

    def group_lc(i):
        if isinstance(i, int):
            return i % HIST_LEN, tb0 + i // HIST_LEN
        return lax.rem(i, HIST_LEN), tb0 + lax.div(i, HIST_LEN)

    def i_desc(i, b):
        l, tb = group_lc(i)
        return pltpu.make_async_copy(
            bd_hbm.at[l, pl.ds(tb * 128, 128)], ibuf[b], isem[b])

    def g_desc(b):
        return pltpu.make_async_copy(tl_hbm.at[gix[b]], rows[b], gsem[b])

    def o_desc(i, b):
        l, tb = group_lc(i)
        return pltpu.make_async_copy(
            stage[b], out_hbm.at[l, :, pl.ds(tb * 128, 128)], osem[b])

    def prep(b):
        for h in range(8):
            iv = ibuf[b][pl.ds(16 * h, 16)]
            gix[b][pl.ds(16 * h, 16)] = jnp.right_shift(iv, 2)
            cov[b][pl.ds(16 * h, 16)] = jnp.left_shift(
                jnp.bitwise_and(iv, 3), 5)

    def extract(b):
        rw = rows[b]
        st = stage[b]
        cv = cov[b]
        iota = _iota16()

        @plsc.parallel_loop(0, 8, unroll=2)
        def _(h):
            rvec = iota + h * 16
            covh = cv[pl.ds(h * 16, 16)]
            vals = [plsc.load_gather(rw, [rvec, covh + d])
                    for d in range(EMBED_DIM)]
            for d in range(EMBED_DIM):
                st[d, pl.ds(h * 16, 16)] = vals[d]

    for b in range(NBUF):
        i_desc(b, b).start()

    def body(g, carry):
        for b in range(NBUF):
            i = g * NBUF + b
            i_desc(i, b).wait()
            prep(b)
            g_desc(b).start()

            @pl.when(g < NG2 - 1)
            def _():
                i_desc(i + NBUF, b).start()

            bp = (b - 1) % NBUF
            ip = i - 1
            if b == 0:
                @pl.when(g >= 1)
                def _():
                    g_desc(bp).wait()

                    @pl.when(g >= 2)
                    def _():
                        o_desc(ip - NBUF, bp).wait()

                    extract(bp)
                    o_desc(ip, bp).start()
            else:
                g_desc(bp).wait()

                @pl.when(g >= 1)
                def _():
                    o_desc(ip - NBUF, bp).wait()

                extract(bp)
                o_desc(ip, bp).start()
        return carry

    lax.fori_loop(0, NG2, body, 0)

    last = N_GROUPS - 1
    g_desc(NBUF - 1).wait()
    o_desc(last - NBUF, NBUF - 1).wait()
    extract(NBUF - 1)
    o_desc(last, NBUF - 1).start()
    for b in range(NBUF):
        o_desc(last - (NBUF - 1) + b, b).wait()


def kernel(batch_data, word_embedding):
    bdT = batch_data.astype(jnp.int32).T
    tabT = word_embedding.T
    tail = word_embedding[VOCAB - 64:].reshape(2048)
    table_lin = _transpose_kernel(tabT, tail).reshape(TL_ROWS, 128)
    out_phys = _gather_kernel(bdT, table_lin)
    return jnp.transpose(out_phys, (2, 0, 1))


# speedup vs baseline: 1.1289x; 1.1289x over previous
import functools

import jax
import jax.numpy as jnp
from jax import lax
from jax.experimental import pallas as pl
from jax.experimental.pallas import tpu as pltpu
from jax.experimental.pallas import tpu_sc as plsc

VOCAB = 1000000
EMBED_DIM = 32
BATCH = 16384
HIST_LEN = 50

NC = 2
NS = 16
NW = NC * NS
NBUF = 4

NT_FULL = 7812
N_K1 = 244
NG1 = N_K1 // NBUF
K1_EXTRA = NT_FULL - NW * N_K1
TL_ROWS = 250000

NTB = BATCH // 128
TB_PER_W = NTB // NW
N_GROUPS = TB_PER_W * HIST_LEN
NG2 = N_GROUPS // NBUF

_MESH = plsc.VectorSubcoreMesh(core_axis_name="c", subcore_axis_name="s")


def _iota16():
    return lax.iota(jnp.int32, 16)


@functools.partial(
    pl.kernel,
    mesh=_MESH,
    out_type=jax.ShapeDtypeStruct((TL_ROWS * 128,), jnp.float32),
    scratch_types=(
        [pltpu.VMEM((32, 128), jnp.float32)] * NBUF
        + [pltpu.VMEM((4096,), jnp.float32)] * NBUF
        + [pltpu.SemaphoreType.DMA] * (2 * NBUF)
    ),
    compiler_params=pltpu.CompilerParams(needs_layout_passes=False),
)
def _transpose_kernel(tab_hbm, tail_hbm, tl_hbm, *bufs):
    tiles = bufs[:NBUF]
    stage = bufs[NBUF:2 * NBUF]
    rsem = bufs[2 * NBUF:3 * NBUF]
    wsem = bufs[3 * NBUF:]

    wid = lax.axis_index("s") * NC + lax.axis_index("c")

    def r_desc(t_v, b):
        return pltpu.make_async_copy(
            tab_hbm.at[:, pl.ds(t_v * 128, 128)], tiles[b], rsem[b])

    def w_desc(t_v, b):
        return pltpu.make_async_copy(
            stage[b], tl_hbm.at[pl.ds(t_v * 4096, 4096)], wsem[b])

    def extract(b):
        tl = tiles[b]
        st = stage[b]
        base = _iota16() * 32
        for g in range(8):
            tgt = base + g * 512
            vals = [tl[d, pl.ds(g * 16, 16)] for d in range(EMBED_DIM)]
            for d in range(EMBED_DIM):
                plsc.store_scatter(st, [tgt + d], vals[d])

    def tv(i):
        return wid + NW * i

    for b in range(NBUF):
        r_desc(tv(b), b).start()

    def body(g, carry):
        for b in range(NBUF):
            i = g * NBUF + b
            r_desc(tv(i), b).wait()

            @pl.when(g >= 1)
            def _():
                w_desc(tv(i - NBUF), b).wait()

            extract(b)
            w_desc(tv(i), b).start()

            @pl.when(g < NG1 - 1)
            def _():
                r_desc(tv(i + NBUF), b).start()
        return carry

    lax.fori_loop(0, NG1, body, 0)

    t_extra = NW * N_K1 + wid
    w_desc(tv(N_K1 - NBUF), 0).wait()

    @pl.when(wid < K1_EXTRA)
    def _():
        r_desc(t_extra, 0).start()
        r_desc(t_extra, 0).wait()

    extract(0)

    @pl.when(wid < K1_EXTRA)
    def _():
        w_desc(t_extra, 0).start()
        w_desc(t_extra, 0).wait()

    @pl.when(wid == K1_EXTRA)
    def _():
        pltpu.sync_copy(tail_hbm, tl_hbm.at[pl.ds((TL_ROWS - 16) * 128, 2048)])

    for b in range(1, NBUF):
        w_desc(tv(N_K1 - NBUF + b), b).wait()


@functools.partial(
    pl.kernel,
    mesh=_MESH,
    out_type=jax.ShapeDtypeStruct((HIST_LEN, EMBED_DIM, BATCH), jnp.float32),
    scratch_types=(
        [pltpu.VMEM((128,), jnp.int32)] * NBUF
        + [pltpu.VMEM((128,), jnp.int32)] * NBUF
        + [pltpu.VMEM((128,), jnp.int32)] * NBUF
        + [pltpu.VMEM((128, 128), jnp.float32)] * NBUF
        + [pltpu.VMEM((32, 128), jnp.float32)] * NBUF
        + [pltpu.SemaphoreType.DMA] * (3 * NBUF)
    ),
    compiler_params=pltpu.CompilerParams(needs_layout_passes=False),
)
def _gather_kernel(bd_hbm, tl_hbm, out_hbm, *bufs):
    ibuf = bufs[:NBUF]
    gix = bufs[NBUF:2 * NBUF]
    cov = bufs[2 * NBUF:3 * NBUF]
    rows = bufs[3 * NBUF:4 * NBUF]
    stage = bufs[4 * NBUF:5 * NBUF]
    isem = bufs[5 * NBUF:6 * NBUF]
    gsem = bufs[6 * NBUF:7 * NBUF]
    osem = bufs[7 * NBUF:]

    wid = lax.axis_index("s") * NC + lax.axis_index("c")
    tb0 = wid * TB_PER_W

    def group_lc(i):
        if isinstance(i, int):
            return i % HIST_LEN, tb0 + i // HIST_LEN
        return lax.rem(i, HIST_LEN), tb0 + lax.div(i, HIST_LEN)

    def i_desc(i, b):
        l, tb = group_lc(i)
        return pltpu.make_async_copy(
            bd_hbm.at[l, pl.ds(tb * 128, 128)], ibuf[b], isem[b])

    def g_desc(b):
        return pltpu.make_async_copy(tl_hbm.at[gix[b]], rows[b], gsem[b])

    def o_desc(i, b):
        l, tb = group_lc(i)
        return pltpu.make_async_copy(
            stage[b], out_hbm.at[l, :, pl.ds(tb * 128, 128)], osem[b])

    def prep(b):
        for h in range(8):
            iv = ibuf[b][pl.ds(16 * h, 16)]
            gix[b][pl.ds(16 * h, 16)] = jnp.right_shift(iv, 2)
            cov[b][pl.ds(16 * h, 16)] = jnp.left_shift(
                jnp.bitwise_and(iv, 3), 5)

    def extract(b):
        rw = rows[b]
        st = stage[b]
        cv = cov[b]
        iota = _iota16()
        for h in range(8):
            rvec = iota + h * 16
            covh = cv[pl.ds(h * 16, 16)]
            vals = [plsc.load_gather(rw, [rvec, covh + d])
                    for d in range(EMBED_DIM)]
            for d in range(EMBED_DIM):
                st[d, pl.ds(h * 16, 16)] = vals[d]

    for b in range(NBUF):
        i_desc(b, b).start()

    def body(g, carry):
        for b in range(NBUF):
            i = g * NBUF + b
            i_desc(i, b).wait()
            prep(b)
            g_desc(b).start()

            @pl.when(g < NG2 - 1)
            def _():
                i_desc(i + NBUF, b).start()

            bp = (b - 1) % NBUF
            ip = i - 1
            if b == 0:
                @pl.when(g >= 1)
                def _():
                    g_desc(bp).wait()

                    @pl.when(g >= 2)
                    def _():
                        o_desc(ip - NBUF, bp).wait()

                    extract(bp)
                    o_desc(ip, bp).start()
            else:
                g_desc(bp).wait()

                @pl.when(g >= 1)
                def _():
                    o_desc(ip - NBUF, bp).wait()

                extract(bp)
                o_desc(ip, bp).start()
        return carry

    lax.fori_loop(0, NG2, body, 0)

    last = N_GROUPS - 1
    g_desc(NBUF - 1).wait()
    o_desc(last - NBUF, NBUF - 1).wait()
    extract(NBUF - 1)
    o_desc(last, NBUF - 1).start()
    for b in range(NBUF):
        o_desc(last - (NBUF - 1) + b, b).wait()


def kernel(batch_data, word_embedding):
    bdT = batch_data.astype(jnp.int32).T
    tabT = word_embedding.T
    tail = word_embedding[VOCAB - 64:].reshape(2048)
    table_lin = _transpose_kernel(tabT, tail).reshape(TL_ROWS, 128)
    out_phys = _gather_kernel(bdT, table_lin)
    return jnp.transpose(out_phys, (2, 0, 1))


# speedup vs baseline: 1.1290x; 1.0001x over previous
import functools

import jax
import jax.numpy as jnp
from jax import lax
from jax.experimental import pallas as pl
from jax.experimental.pallas import tpu as pltpu
from jax.experimental.pallas import tpu_sc as plsc

VOCAB = 1000000
EMBED_DIM = 32
BATCH = 16384
HIST_LEN = 50

NC = 2
NS = 16
NW = NC * NS
NBUF = 4

NT_FULL = 7812
N_K1 = 244
NG1 = N_K1 // NBUF
K1_EXTRA = NT_FULL - NW * N_K1
TL_ROWS = 250000

NTB = BATCH // 128
TB_PER_W = NTB // NW
N_GROUPS = TB_PER_W * HIST_LEN
NG2 = N_GROUPS // NBUF

_MESH = plsc.VectorSubcoreMesh(core_axis_name="c", subcore_axis_name="s")


def _iota16():
    return lax.iota(jnp.int32, 16)


@functools.partial(
    pl.kernel,
    mesh=_MESH,
    out_type=jax.ShapeDtypeStruct((TL_ROWS * 128,), jnp.float32),
    scratch_types=(
        [pltpu.VMEM((32, 128), jnp.float32)] * NBUF
        + [pltpu.VMEM((4096,), jnp.float32)] * NBUF
        + [pltpu.SemaphoreType.DMA] * (2 * NBUF)
    ),
    compiler_params=pltpu.CompilerParams(
        needs_layout_passes=False, disable_bounds_checks=True),
)
def _transpose_kernel(tab_hbm, tail_hbm, tl_hbm, *bufs):
    tiles = bufs[:NBUF]
    stage = bufs[NBUF:2 * NBUF]
    rsem = bufs[2 * NBUF:3 * NBUF]
    wsem = bufs[3 * NBUF:]

    wid = lax.axis_index("s") * NC + lax.axis_index("c")

    def r_desc(t_v, b):
        return pltpu.make_async_copy(
            tab_hbm.at[:, pl.ds(t_v * 128, 128)], tiles[b], rsem[b])

    def w_desc(t_v, b):
        return pltpu.make_async_copy(
            stage[b], tl_hbm.at[pl.ds(t_v * 4096, 4096)], wsem[b])

    def extract(b):
        tl = tiles[b]
        st = stage[b]
        base = _iota16() * 32
        for g in range(8):
            tgt = base + g * 512
            vals = [tl[d, pl.ds(g * 16, 16)] for d in range(EMBED_DIM)]
            for d in range(EMBED_DIM):
                plsc.store_scatter(st, [tgt + d], vals[d])

    def tv(i):
        return wid + NW * i

    for b in range(NBUF):
        r_desc(tv(b), b).start()

    def body(g, carry):
        for b in range(NBUF):
            i = g * NBUF + b
            r_desc(tv(i), b).wait()

            @pl.when(g >= 1)
            def _():
                w_desc(tv(i - NBUF), b).wait()

            extract(b)
            w_desc(tv(i), b).start()

            @pl.when(g < NG1 - 1)
            def _():
                r_desc(tv(i + NBUF), b).start()
        return carry

    lax.fori_loop(0, NG1, body, 0)

    t_extra = NW * N_K1 + wid
    w_desc(tv(N_K1 - NBUF), 0).wait()

    @pl.when(wid < K1_EXTRA)
    def _():
        r_desc(t_extra, 0).start()
        r_desc(t_extra, 0).wait()

    extract(0)

    @pl.when(wid < K1_EXTRA)
    def _():
        w_desc(t_extra, 0).start()
        w_desc(t_extra, 0).wait()

    @pl.when(wid == K1_EXTRA)
    def _():
        pltpu.sync_copy(tail_hbm, tl_hbm.at[pl.ds((TL_ROWS - 16) * 128, 2048)])

    for b in range(1, NBUF):
        w_desc(tv(N_K1 - NBUF + b), b).wait()


@functools.partial(
    pl.kernel,
    mesh=_MESH,
    out_type=jax.ShapeDtypeStruct((HIST_LEN, EMBED_DIM, BATCH), jnp.float32),
    scratch_types=(
        [pltpu.VMEM((128,), jnp.int32)] * NBUF
        + [pltpu.VMEM((128,), jnp.int32)] * NBUF
        + [pltpu.VMEM((128,), jnp.int32)] * NBUF
        + [pltpu.VMEM((128, 128), jnp.float32)] * NBUF
        + [pltpu.VMEM((32, 128), jnp.float32)] * NBUF
        + [pltpu.SemaphoreType.DMA] * (3 * NBUF)
    ),
    compiler_params=pltpu.CompilerParams(
        needs_layout_passes=False, disable_bounds_checks=True),
)
def _gather_kernel(bd_hbm, tl_hbm, out_hbm, *bufs):
    ibuf = bufs[:NBUF]
    gix = bufs[NBUF:2 * NBUF]
    cov = bufs[2 * NBUF:3 * NBUF]
    rows = bufs[3 * NBUF:4 * NBUF]
    stage = bufs[4 * NBUF:5 * NBUF]
    isem = bufs[5 * NBUF:6 * NBUF]
    gsem = bufs[6 * NBUF:7 * NBUF]
    osem = bufs[7 * NBUF:]

    wid = lax.axis_index("s") * NC + lax.axis_index("c")
    tb0 = wid * TB_PER_W

    def group_lc(i):
        if isinstance(i, int):
            return i % HIST_LEN, tb0 + i // HIST_LEN
        return lax.rem(i, HIST_LEN), tb0 + lax.div(i, HIST_LEN)

    def i_desc(i, b):
        l, tb = group_lc(i)
        return pltpu.make_async_copy(
            bd_hbm.at[l, pl.ds(tb * 128, 128)], ibuf[b], isem[b])

    def g_desc(b):
        return pltpu.make_async_copy(tl_hbm.at[gix[b]], rows[b], gsem[b])

    def o_desc(i, b):
        l, tb = group_lc(i)
        return pltpu.make_async_copy(
            stage[b], out_hbm.at[l, :, pl.ds(tb * 128, 128)], osem[b])

    def prep(b):
        for h in range(8):
            iv = ibuf[b][pl.ds(16 * h, 16)]
            gix[b][pl.ds(16 * h, 16)] = jnp.right_shift(iv, 2)
            cov[b][pl.ds(16 * h, 16)] = jnp.left_shift(
                jnp.bitwise_and(iv, 3), 5)

    def extract(b):
        rw = rows[b]
        st = stage[b]
        cv = cov[b]
        iota = _iota16()
        for h in range(8):
            rvec = iota + h * 16
            covh = cv[pl.ds(h * 16, 16)]
            vals = [plsc.load_gather(rw, [rvec, covh + d])
                    for d in range(EMBED_DIM)]
            for d in range(EMBED_DIM):
                st[d, pl.ds(h * 16, 16)] = vals[d]

    for b in range(NBUF):
        i_desc(b, b).start()

    def body(g, carry):
        for b in range(NBUF):
            i = g * NBUF + b
            i_desc(i, b).wait()
            prep(b)
            g_desc(b).start()

            @pl.when(g < NG2 - 1)
            def _():
                i_desc(i + NBUF, b).start()

            bp = (b - 1) % NBUF
            ip = i - 1
            if b == 0:
                @pl.when(g >= 1)
                def _():
                    g_desc(bp).wait()

                    @pl.when(g >= 2)
                    def _():
                        o_desc(ip - NBUF, bp).wait()

                    extract(bp)
                    o_desc(ip, bp).start()
            else:
                g_desc(bp).wait()

                @pl.when(g >= 1)
                def _():
                    o_desc(ip - NBUF, bp).wait()

                extract(bp)
                o_desc(ip, bp).start()
        return carry

    lax.fori_loop(0, NG2, body, 0)

    last = N_GROUPS - 1
    g_desc(NBUF - 1).wait()
    o_desc(last - NBUF, NBUF - 1).wait()
    extract(NBUF - 1)
    o_desc(last, NBUF - 1).start()
    for b in range(NBUF):
        o_desc(last - (NBUF - 1) + b, b).wait()


def kernel(batch_data, word_embedding):
    bdT = batch_data.astype(jnp.int32).T
    tabT = word_embedding.T
    tail = word_embedding[VOCAB - 64:].reshape(2048)
    table_lin = _transpose_kernel(tabT, tail).reshape(TL_ROWS, 128)
    out_phys = _gather_kernel(bdT, table_lin)
    return jnp.transpose(out_phys, (2, 0, 1))
